# Initial kernel scaffold; baseline (speedup 1.0000x reference)
import jax
import jax.numpy as jnp
from jax.experimental import pallas as pl


def kernel(x, control_points):
    raise NotImplementedError("write your pallas kernel here")


# speedup vs baseline: 2164.5434x; 2164.5434x over previous
import numpy as np
import jax
import jax.numpy as jnp
from jax.experimental import pallas as pl
from jax.experimental.pallas import tpu as pltpu

_GRID_SIZE = 32
_DEGREE = 3
_NUM_CP = _GRID_SIZE - 1 + _DEGREE + 1
_NUM_SEG = _NUM_CP - _DEGREE

_KNOTS = np.asarray(
    jnp.linspace(0.0, 1.0, _GRID_SIZE + 2 * _DEGREE).astype(jnp.float32))
_DMIN = float(_KNOTS[_DEGREE])
_DMAX = float(_KNOTS[-_DEGREE - 1])
_DEN = float(np.float32(np.float32(_DMAX - _DMIN) + np.float32(1e-6)))

_PSI = (np.array([[1., 4., 1., 0.],
                  [-3., 0., 3., 0.],
                  [3., -6., 3., 0.],
                  [-1., 3., -3., 1.]], dtype=np.float64) / 6.0).astype(np.float32)

_BBLK = 512


def _spline_block(x_ref, cpt_ref, out_ref):
    x = x_ref[...]
    xc = jnp.clip(x, _DMIN, _DMAX)
    u = jnp.clip((xc - _DMIN) / _DEN, 0.0, 1.0)

    cpt = cpt_ref[...]
    g = []
    for p in range(4):
        acc = None
        for j in range(4):
            c = float(_PSI[p, j])
            if c == 0.0:
                continue
            term = c * cpt[j:j + _NUM_SEG, :]
            acc = term if acc is None else acc + term
        g.append(acc)

    r = [jnp.broadcast_to(g[p][0:1, :], x.shape) for p in range(4)]
    for s in range(1, _NUM_SEG):
        cond = xc > _KNOTS[s + 2]
        for p in range(4):
            r[p] = jnp.where(cond, g[p][s:s + 1, :], r[p])

    out_ref[...] = r[0] + u * (r[1] + u * (r[2] + u * r[3]))


def kernel(x, control_points):
    b, d = x.shape
    cpt = control_points.T
    grid = b // _BBLK
    return pl.pallas_call(
        _spline_block,
        grid=(grid,),
        in_specs=[
            pl.BlockSpec((_BBLK, d), lambda i: (i, 0)),
            pl.BlockSpec((_NUM_CP, d), lambda i: (0, 0)),
        ],
        out_specs=pl.BlockSpec((_BBLK, d), lambda i: (i, 0)),
        out_shape=jax.ShapeDtypeStruct((b, d), jnp.float32),
        compiler_params=pltpu.CompilerParams(
            dimension_semantics=("arbitrary",),
        ),
    )(x, cpt)
